# baseline (speedup 1.0000x reference)
import functools

import jax
import jax.numpy as jnp
from jax import lax
from jax.experimental import pallas as pl
from jax.experimental.pallas import tpu as pltpu
from jax.experimental.pallas import tpu_sc as plsc

N = 10000
E = 640000
H = 128
NC, NS = 2, 16
NW = NC * NS
EPW = E // NW
L = 16

_SC_MESH = dict(core_axis_name="c", subcore_axis_name="s",
                num_cores=NC, num_subcores=NS)


def _make_seg_sum(with_count: bool):
    out_type = [jax.ShapeDtypeStruct((NW, N), jnp.float32)]
    scratch = [
        pltpu.VMEM((N,), jnp.float32),
        pltpu.VMEM((EPW,), jnp.int32),
        pltpu.VMEM((EPW,), jnp.int32),
        pltpu.VMEM((N,), jnp.float32),
    ]
    if with_count:
        out_type.append(jax.ShapeDtypeStruct((NW, N), jnp.float32))
        scratch.append(pltpu.VMEM((N,), jnp.float32))
    scratch += [pltpu.SemaphoreType.DMA] * 3

    UNROLL = 5
    assert (EPW // L) % UNROLL == 0

    @functools.partial(
        pl.kernel,
        out_type=out_type,
        mesh=plsc.VectorSubcoreMesh(**_SC_MESH),
        scratch_types=scratch,
        compiler_params=pltpu.CompilerParams(needs_layout_passes=False),
    )
    def seg(vals_hbm, src_hbm, dst_hbm, acc_out, *rest):
        if with_count:
            cnt_out, vals_v, src_v, dst_v, acc_v, cnt_v, s0, s1, s2 = rest
        else:
            vals_v, src_v, dst_v, acc_v, s0, s1, s2 = rest
        wid = lax.axis_index("s") * NC + lax.axis_index("c")
        base = wid * EPW
        c0 = pltpu.async_copy(vals_hbm, vals_v, s0)
        c1 = pltpu.async_copy(src_hbm.at[pl.ds(base, EPW)], src_v, s1)
        c2 = pltpu.async_copy(dst_hbm.at[pl.ds(base, EPW)], dst_v, s2)

        zeros = jnp.zeros((L,), jnp.float32)

        def zbody(i, carry):
            acc_v[pl.ds(i * L, L)] = zeros
            if with_count:
                cnt_v[pl.ds(i * L, L)] = zeros
            return carry

        lax.fori_loop(0, N // L, zbody, 0)
        c0.wait()
        c1.wait()
        c2.wait()

        ones = jnp.ones((L,), jnp.float32)

        def ebody(j, carry):
            for k in range(UNROLL):
                i = j * UNROLL + k
                si = src_v[pl.ds(i * L, L)]
                di = dst_v[pl.ds(i * L, L)]
                v = plsc.load_gather(vals_v, [si])
                plsc.addupdate_scatter(acc_v, [di], v)
                if with_count:
                    plsc.addupdate_scatter(cnt_v, [di], ones)
            return carry

        lax.fori_loop(0, EPW // L // UNROLL, ebody, 0)

        pltpu.sync_copy(acc_v, acc_out.at[wid])
        if with_count:
            pltpu.sync_copy(cnt_v, cnt_out.at[wid])

    return seg


@functools.lru_cache(maxsize=None)
def _get_seg_kernel(with_count: bool):
    return _make_seg_sum(with_count)


def _seg_sum_cnt(vals, src, dst):
    return _get_seg_kernel(True)(vals, src, dst)


def _seg_sum(vals, src, dst):
    return _get_seg_kernel(False)(vals, src, dst)


def _node_phase(x_row, acc_p, cnt_p, ws1c, wn1c, b1c, ws2c, wn2c):
    def body(x_ref, acc_ref, cnt_ref, ws1_ref, wn1_ref, b1_ref, ws2_ref,
             wn2_ref, s_ref, u_ref, deg_ref):
        deg = jnp.maximum(jnp.sum(cnt_ref[...], axis=0, keepdims=True), 1.0)
        m1 = jnp.sum(acc_ref[...], axis=0, keepdims=True) / deg
        h1 = jnp.tanh(ws1_ref[...] * x_ref[...] + wn1_ref[...] * m1
                      + b1_ref[...])
        s_ref[...] = jnp.dot(ws2_ref[...], h1,
                             preferred_element_type=jnp.float32)
        u_ref[...] = jnp.dot(wn2_ref[...], h1,
                             preferred_element_type=jnp.float32)
        deg_ref[...] = deg

    return pl.pallas_call(
        body,
        out_shape=[jax.ShapeDtypeStruct((1, N), jnp.float32),
                   jax.ShapeDtypeStruct((1, N), jnp.float32),
                   jax.ShapeDtypeStruct((1, N), jnp.float32)],
    )(x_row, acc_p, cnt_p, ws1c, wn1c, b1c, ws2c, wn2c)


def _o1_phase(s, sum2_p, deg, b2):
    def body(s_ref, p_ref, deg_ref, b2_ref, o1_ref, h2_ref):
        m2 = jnp.sum(p_ref[...], axis=0, keepdims=True) / deg_ref[...]
        o1 = s_ref[...] + m2 + b2_ref[0, 0]
        o1_ref[...] = o1
        h2_ref[...] = jnp.tanh(o1)

    return pl.pallas_call(
        body,
        in_specs=[
            pl.BlockSpec(memory_space=pltpu.VMEM),
            pl.BlockSpec(memory_space=pltpu.VMEM),
            pl.BlockSpec(memory_space=pltpu.VMEM),
            pl.BlockSpec(memory_space=pltpu.SMEM),
        ],
        out_shape=[jax.ShapeDtypeStruct((1, N), jnp.float32),
                   jax.ShapeDtypeStruct((1, N), jnp.float32)],
    )(s, sum2_p, deg, b2)


R_TC = 6160
R_SC = N - R_TC
ROWS_PT = R_SC // NW
CH = 8
NRC = ROWS_PT // CH
CW = 1664
NCC = 6
SC_COLS = CW * NCC
CWV = CW // L


@functools.lru_cache(maxsize=None)
def _get_sc_matvec():
    @functools.partial(
        pl.kernel,
        out_type=[jax.ShapeDtypeStruct((NW, N), jnp.float32)],
        mesh=plsc.VectorSubcoreMesh(**_SC_MESH),
        scratch_types=[
            pltpu.VMEM((ROWS_PT * L,), jnp.float32),
            pltpu.VMEM((CH, CW), jnp.float32),
            pltpu.VMEM((CH, CW), jnp.float32),
            pltpu.VMEM((N,), jnp.float32),
            pltpu.SemaphoreType.DMA,
            pltpu.SemaphoreType.DMA,
            pltpu.SemaphoreType.DMA,
        ],
        compiler_params=pltpu.CompilerParams(needs_layout_passes=False),
    )
    def scmv(h2_hbm, w3_hbm, out_hbm, h2_v, bufa, bufb, acc_v, sa, sb, sh):
        wid = lax.axis_index("s") * NC + lax.axis_index("c")
        row0 = R_TC + wid * ROWS_PT
        hcp = pltpu.async_copy(h2_hbm.at[pl.ds(row0 * L, ROWS_PT * L)], h2_v,
                               sh)
        zeros = jnp.zeros((L,), jnp.float32)

        def zbody(i, carry):
            acc_v[pl.ds(i * L, L)] = zeros
            return carry

        lax.fori_loop(0, N // L, zbody, 0)
        hcp.wait()

        bufs = (bufa, bufb)
        sems = (sa, sb)
        chunks = [(rc, cc) for rc in range(NRC) for cc in range(NCC)]

        def start(k):
            rc, cc = chunks[k]
            return pltpu.async_copy(
                w3_hbm.at[pl.ds(row0 + rc * CH, CH), pl.ds(cc * CW, CW)],
                bufs[k % 2], sems[k % 2])

        def compute(rc, cc, buf):
            colbase = cc * CW

            def rbody(r, carry):
                hvec = h2_v[pl.ds((rc * CH + r) * L, L)]

                def cbody(j, carry2):
                    wv = buf[r, pl.ds(j * L, L)]
                    off = colbase + j * L
                    acc_v[pl.ds(off, L)] = acc_v[pl.ds(off, L)] + hvec * wv
                    return carry2

                lax.fori_loop(0, CWV, cbody, 0)
                return carry

            lax.fori_loop(0, CH, rbody, 0)

        pending = start(0)
        for k in range(len(chunks)):
            nxt = start(k + 1) if k + 1 < len(chunks) else None
            pending.wait()
            rc, cc = chunks[k]
            compute(rc, cc, bufs[k % 2])
            pending = nxt

        pltpu.sync_copy(acc_v, out_hbm.at[wid])

    return scmv


def _sc_matvec(h2, w3):
    h2_rep = jnp.broadcast_to(h2.reshape(N, 1), (N, L)).reshape(N * L)
    return _get_sc_matvec()(h2_rep, w3)


def _tc_matvec(h2, w3, b3_row, block_rows=176):
    nsteps = R_TC // block_rows
    h2_3d = h2[:, :R_TC].reshape(nsteps, 1, block_rows)

    def body(h2_ref, w3_ref, b3_ref, o2_ref):
        i = pl.program_id(0)

        @pl.when(i == 0)
        def _():
            o2_ref[...] = b3_ref[...]

        o2_ref[...] += jnp.dot(h2_ref[...].reshape(1, block_rows), w3_ref[...],
                               preferred_element_type=jnp.float32)

    return pl.pallas_call(
        body,
        grid=(nsteps,),
        in_specs=[
            pl.BlockSpec((1, 1, block_rows), lambda i: (i, 0, 0)),
            pl.BlockSpec((block_rows, N), lambda i: (i, 0)),
            pl.BlockSpec((1, N), lambda i: (0, 0)),
        ],
        out_specs=pl.BlockSpec((1, N), lambda i: (0, 0)),
        out_shape=jax.ShapeDtypeStruct((1, N), jnp.float32),
    )(h2_3d, w3, b3_row)


def _combine(o2_tc, sc_part, h2_sc, w3_tail):
    def body(t_ref, p_ref, h_ref, wt_ref, o_ref):
        o_ref[...] = t_ref[...] + jnp.sum(p_ref[...], axis=0, keepdims=True)
        tail = jnp.dot(h_ref[...], wt_ref[...],
                       preferred_element_type=jnp.float32)
        o_ref[:, SC_COLS:N] += tail

    return pl.pallas_call(
        body,
        out_shape=jax.ShapeDtypeStruct((1, N), jnp.float32),
    )(o2_tc, sc_part, h2_sc, w3_tail)


def kernel(node_features, edge_index, w_self1, w_neigh1, b1, w_self2,
           w_neigh2, b2, w3, b3):
    x_flat = node_features.reshape(N)
    src = edge_index[0]
    dst = edge_index[1]

    sum1_p, cnt_p = _seg_sum_cnt(x_flat, src, dst)

    x_row = node_features.reshape(1, N)
    ws1c = w_self1.reshape(H, 1)
    wn1c = w_neigh1.reshape(H, 1)
    b1c = b1.reshape(H, 1)
    ws2c = w_self2.reshape(1, H)
    wn2c = w_neigh2.reshape(1, H)
    s, u, deg = _node_phase(x_row, sum1_p, cnt_p, ws1c, wn1c, b1c, ws2c, wn2c)

    (sum2_p,) = _seg_sum(u.reshape(N), src, dst)

    o1, h2 = _o1_phase(s, sum2_p, deg, b2.reshape(1, 1))
    (sc_part,) = _sc_matvec(h2, w3)
    o2_tc = _tc_matvec(h2, w3, b3.reshape(1, N))
    o2 = _combine(o2_tc, sc_part, h2[:, R_TC:], w3[R_TC:, SC_COLS:])
    return (o1, o2)


# speedup vs baseline: 1.1881x; 1.1881x over previous
import functools

import jax
import jax.numpy as jnp
from jax import lax
from jax.experimental import pallas as pl
from jax.experimental.pallas import tpu as pltpu
from jax.experimental.pallas import tpu_sc as plsc

N = 10000
E = 640000
H = 128
NC, NS = 2, 16
NW = NC * NS
EPW = E // NW
L = 16

_SC_MESH = dict(core_axis_name="c", subcore_axis_name="s",
                num_cores=NC, num_subcores=NS)


def _make_seg_sum(with_count: bool):
    out_type = [jax.ShapeDtypeStruct((NW, N), jnp.float32)]
    scratch = [
        pltpu.VMEM((N,), jnp.float32),
        pltpu.VMEM((EPW,), jnp.int32),
        pltpu.VMEM((EPW,), jnp.int32),
        pltpu.VMEM((N,), jnp.float32),
    ]
    if with_count:
        out_type.append(jax.ShapeDtypeStruct((NW, N), jnp.float32))
        scratch.append(pltpu.VMEM((N,), jnp.float32))
    scratch += [pltpu.SemaphoreType.DMA] * 3

    UNROLL = 5
    assert (EPW // L) % UNROLL == 0

    @functools.partial(
        pl.kernel,
        out_type=out_type,
        mesh=plsc.VectorSubcoreMesh(**_SC_MESH),
        scratch_types=scratch,
        compiler_params=pltpu.CompilerParams(needs_layout_passes=False),
    )
    def seg(vals_hbm, src_hbm, dst_hbm, acc_out, *rest):
        if with_count:
            cnt_out, vals_v, src_v, dst_v, acc_v, cnt_v, s0, s1, s2 = rest
        else:
            vals_v, src_v, dst_v, acc_v, s0, s1, s2 = rest
        wid = lax.axis_index("s") * NC + lax.axis_index("c")
        base = wid * EPW
        c0 = pltpu.async_copy(vals_hbm, vals_v, s0)
        c1 = pltpu.async_copy(src_hbm.at[pl.ds(base, EPW)], src_v, s1)
        c2 = pltpu.async_copy(dst_hbm.at[pl.ds(base, EPW)], dst_v, s2)

        zeros = jnp.zeros((L,), jnp.float32)

        def zbody(i, carry):
            acc_v[pl.ds(i * L, L)] = zeros
            if with_count:
                cnt_v[pl.ds(i * L, L)] = zeros
            return carry

        lax.fori_loop(0, N // L, zbody, 0)
        c0.wait()
        c1.wait()
        c2.wait()

        ones = jnp.ones((L,), jnp.float32)

        def ebody(j, carry):
            for k in range(UNROLL):
                i = j * UNROLL + k
                si = src_v[pl.ds(i * L, L)]
                di = dst_v[pl.ds(i * L, L)]
                v = plsc.load_gather(vals_v, [si])
                plsc.addupdate_scatter(acc_v, [di], v)
                if with_count:
                    plsc.addupdate_scatter(cnt_v, [di], ones)
            return carry

        lax.fori_loop(0, EPW // L // UNROLL, ebody, 0)

        pltpu.sync_copy(acc_v, acc_out.at[wid])
        if with_count:
            pltpu.sync_copy(cnt_v, cnt_out.at[wid])

    return seg


@functools.lru_cache(maxsize=None)
def _get_seg_kernel(with_count: bool):
    return _make_seg_sum(with_count)


def _seg_sum_cnt(vals, src, dst):
    return _get_seg_kernel(True)(vals, src, dst)


def _seg_sum(vals, src, dst):
    return _get_seg_kernel(False)(vals, src, dst)


def _node_phase(x_row, acc_p, cnt_p, ws1c, wn1c, b1c, ws2c, wn2c):
    def body(x_ref, acc_ref, cnt_ref, ws1_ref, wn1_ref, b1_ref, ws2_ref,
             wn2_ref, s_ref, u_ref, deg_ref):
        deg = jnp.maximum(jnp.sum(cnt_ref[...], axis=0, keepdims=True), 1.0)
        m1 = jnp.sum(acc_ref[...], axis=0, keepdims=True) / deg
        h1 = jnp.tanh(ws1_ref[...] * x_ref[...] + wn1_ref[...] * m1
                      + b1_ref[...])
        s_ref[...] = jnp.dot(ws2_ref[...], h1,
                             preferred_element_type=jnp.float32)
        u_ref[...] = jnp.dot(wn2_ref[...], h1,
                             preferred_element_type=jnp.float32)
        deg_ref[...] = deg

    return pl.pallas_call(
        body,
        out_shape=[jax.ShapeDtypeStruct((1, N), jnp.float32),
                   jax.ShapeDtypeStruct((1, N), jnp.float32),
                   jax.ShapeDtypeStruct((1, N), jnp.float32)],
    )(x_row, acc_p, cnt_p, ws1c, wn1c, b1c, ws2c, wn2c)


def _o1_phase(s, sum2_p, deg, b2):
    def body(s_ref, p_ref, deg_ref, b2_ref, o1_ref, h2_ref):
        m2 = jnp.sum(p_ref[...], axis=0, keepdims=True) / deg_ref[...]
        o1 = s_ref[...] + m2 + b2_ref[0, 0]
        o1_ref[...] = o1
        h2_ref[...] = jnp.tanh(o1)

    return pl.pallas_call(
        body,
        in_specs=[
            pl.BlockSpec(memory_space=pltpu.VMEM),
            pl.BlockSpec(memory_space=pltpu.VMEM),
            pl.BlockSpec(memory_space=pltpu.VMEM),
            pl.BlockSpec(memory_space=pltpu.SMEM),
        ],
        out_shape=[jax.ShapeDtypeStruct((1, N), jnp.float32),
                   jax.ShapeDtypeStruct((1, N), jnp.float32)],
    )(s, sum2_p, deg, b2)


R_TC = 6160
R_SC = N - R_TC
ROWS_PT = R_SC // NW
CH = 8
NRC = ROWS_PT // CH
CW = 1664
NCC = 6
SC_COLS = CW * NCC
CWV = CW // L


@functools.lru_cache(maxsize=None)
def _get_sc_matvec():
    @functools.partial(
        pl.kernel,
        out_type=[jax.ShapeDtypeStruct((NW, N), jnp.float32)],
        mesh=plsc.VectorSubcoreMesh(**_SC_MESH),
        scratch_types=[
            pltpu.VMEM((ROWS_PT * L,), jnp.float32),
            pltpu.VMEM((CH, CW), jnp.float32),
            pltpu.VMEM((CH, CW), jnp.float32),
            pltpu.VMEM((N,), jnp.float32),
            pltpu.SemaphoreType.DMA,
            pltpu.SemaphoreType.DMA,
            pltpu.SemaphoreType.DMA,
        ],
        compiler_params=pltpu.CompilerParams(needs_layout_passes=False),
    )
    def scmv(h2_hbm, w3_hbm, out_hbm, h2_v, bufa, bufb, acc_v, sa, sb, sh):
        wid = lax.axis_index("s") * NC + lax.axis_index("c")
        row0 = R_TC + wid * ROWS_PT
        hcp = pltpu.async_copy(h2_hbm.at[pl.ds(row0 * L, ROWS_PT * L)], h2_v,
                               sh)
        zeros = jnp.zeros((L,), jnp.float32)

        def zbody(i, carry):
            acc_v[pl.ds(i * L, L)] = zeros
            return carry

        lax.fori_loop(0, N // L, zbody, 0)
        hcp.wait()

        bufs = (bufa, bufb)
        sems = (sa, sb)
        chunks = [(rc, cc) for rc in range(NRC) for cc in range(NCC)]

        def start(k):
            rc, cc = chunks[k]
            return pltpu.async_copy(
                w3_hbm.at[pl.ds(row0 + rc * CH, CH), pl.ds(cc * CW, CW)],
                bufs[k % 2], sems[k % 2])

        def compute(rc, cc, buf):
            colbase = cc * CW

            def rbody(r, carry):
                hvec = h2_v[pl.ds((rc * CH + r) * L, L)]

                def cbody(jj, carry2):
                    for t in range(8):
                        j = jj * 8 + t
                        wv = buf[r, pl.ds(j * L, L)]
                        plsc.addupdate(acc_v.at[pl.ds(colbase + j * L, L)],
                                       hvec * wv)
                    return carry2

                lax.fori_loop(0, CWV // 8, cbody, 0)
                return carry

            lax.fori_loop(0, CH, rbody, 0)

        pending = start(0)
        for k in range(len(chunks)):
            nxt = start(k + 1) if k + 1 < len(chunks) else None
            pending.wait()
            rc, cc = chunks[k]
            compute(rc, cc, bufs[k % 2])
            pending = nxt

        pltpu.sync_copy(acc_v, out_hbm.at[wid])

    return scmv


def _sc_matvec(h2, w3):
    h2_rep = jnp.broadcast_to(h2.reshape(N, 1), (N, L)).reshape(N * L)
    return _get_sc_matvec()(h2_rep, w3)


def _tc_matvec(h2, w3, b3_row, block_rows=176):
    nsteps = R_TC // block_rows
    h2_3d = h2[:, :R_TC].reshape(nsteps, 1, block_rows)

    def body(h2_ref, w3_ref, b3_ref, o2_ref):
        i = pl.program_id(0)

        @pl.when(i == 0)
        def _():
            o2_ref[...] = b3_ref[...]

        o2_ref[...] += jnp.dot(h2_ref[...].reshape(1, block_rows), w3_ref[...],
                               preferred_element_type=jnp.float32)

    return pl.pallas_call(
        body,
        grid=(nsteps,),
        in_specs=[
            pl.BlockSpec((1, 1, block_rows), lambda i: (i, 0, 0)),
            pl.BlockSpec((block_rows, N), lambda i: (i, 0)),
            pl.BlockSpec((1, N), lambda i: (0, 0)),
        ],
        out_specs=pl.BlockSpec((1, N), lambda i: (0, 0)),
        out_shape=jax.ShapeDtypeStruct((1, N), jnp.float32),
    )(h2_3d, w3, b3_row)


def _combine(o2_tc, sc_part, h2_sc, w3_tail):
    def body(t_ref, p_ref, h_ref, wt_ref, o_ref):
        o_ref[...] = t_ref[...] + jnp.sum(p_ref[...], axis=0, keepdims=True)
        tail = jnp.dot(h_ref[...], wt_ref[...],
                       preferred_element_type=jnp.float32)
        o_ref[:, SC_COLS:N] += tail

    return pl.pallas_call(
        body,
        out_shape=jax.ShapeDtypeStruct((1, N), jnp.float32),
    )(o2_tc, sc_part, h2_sc, w3_tail)


def kernel(node_features, edge_index, w_self1, w_neigh1, b1, w_self2,
           w_neigh2, b2, w3, b3):
    x_flat = node_features.reshape(N)
    src = edge_index[0]
    dst = edge_index[1]

    sum1_p, cnt_p = _seg_sum_cnt(x_flat, src, dst)

    x_row = node_features.reshape(1, N)
    ws1c = w_self1.reshape(H, 1)
    wn1c = w_neigh1.reshape(H, 1)
    b1c = b1.reshape(H, 1)
    ws2c = w_self2.reshape(1, H)
    wn2c = w_neigh2.reshape(1, H)
    s, u, deg = _node_phase(x_row, sum1_p, cnt_p, ws1c, wn1c, b1c, ws2c, wn2c)

    (sum2_p,) = _seg_sum(u.reshape(N), src, dst)

    o1, h2 = _o1_phase(s, sum2_p, deg, b2.reshape(1, 1))
    (sc_part,) = _sc_matvec(h2, w3)
    o2_tc = _tc_matvec(h2, w3, b3.reshape(1, N))
    o2 = _combine(o2_tc, sc_part, h2[:, R_TC:], w3[R_TC:, SC_COLS:])
    return (o1, o2)


# speedup vs baseline: 2.6913x; 2.2652x over previous
import functools

import jax
import jax.numpy as jnp
from jax import lax
from jax.experimental import pallas as pl
from jax.experimental.pallas import tpu as pltpu
from jax.experimental.pallas import tpu_sc as plsc

N = 10000
E = 640000
H = 128
NC, NS = 2, 16
NW = NC * NS
EPW = E // NW
L = 16

_SC_MESH = dict(core_axis_name="c", subcore_axis_name="s",
                num_cores=NC, num_subcores=NS)


def _make_seg_sum(with_count: bool):
    out_type = [jax.ShapeDtypeStruct((NW, N), jnp.float32)]
    scratch = [
        pltpu.VMEM((N,), jnp.float32),
        pltpu.VMEM((EPW,), jnp.int32),
        pltpu.VMEM((EPW,), jnp.int32),
        pltpu.VMEM((N,), jnp.float32),
    ]
    if with_count:
        out_type.append(jax.ShapeDtypeStruct((NW, N), jnp.float32))
        scratch.append(pltpu.VMEM((N,), jnp.float32))
    scratch += [pltpu.SemaphoreType.DMA] * 3

    UNROLL = 10
    assert (EPW // L) % UNROLL == 0

    @functools.partial(
        pl.kernel,
        out_type=out_type,
        mesh=plsc.VectorSubcoreMesh(**_SC_MESH),
        scratch_types=scratch,
        compiler_params=pltpu.CompilerParams(needs_layout_passes=False),
    )
    def seg(vals_hbm, src_hbm, dst_hbm, acc_out, *rest):
        if with_count:
            cnt_out, vals_v, src_v, dst_v, acc_v, cnt_v, s0, s1, s2 = rest
        else:
            vals_v, src_v, dst_v, acc_v, s0, s1, s2 = rest
        wid = lax.axis_index("s") * NC + lax.axis_index("c")
        base = wid * EPW
        c0 = pltpu.async_copy(vals_hbm, vals_v, s0)
        c1 = pltpu.async_copy(src_hbm.at[pl.ds(base, EPW)], src_v, s1)
        c2 = pltpu.async_copy(dst_hbm.at[pl.ds(base, EPW)], dst_v, s2)

        zeros = jnp.zeros((L,), jnp.float32)

        def zbody(i5, carry):
            for k in range(5):
                i = i5 * 5 + k
                acc_v[pl.ds(i * L, L)] = zeros
                if with_count:
                    cnt_v[pl.ds(i * L, L)] = zeros
            return carry

        lax.fori_loop(0, N // L // 5, zbody, 0)
        c0.wait()
        c1.wait()
        c2.wait()

        ones = jnp.ones((L,), jnp.float32)

        def ebody(j, carry):
            for k in range(UNROLL):
                i = j * UNROLL + k
                si = src_v[pl.ds(i * L, L)]
                di = dst_v[pl.ds(i * L, L)]
                v = plsc.load_gather(vals_v, [si])
                plsc.addupdate_scatter(acc_v, [di], v)
                if with_count:
                    plsc.addupdate_scatter(cnt_v, [di], ones)
            return carry

        lax.fori_loop(0, EPW // L // UNROLL, ebody, 0)

        pltpu.sync_copy(acc_v, acc_out.at[wid])
        if with_count:
            pltpu.sync_copy(cnt_v, cnt_out.at[wid])

    return seg


@functools.lru_cache(maxsize=None)
def _get_seg_kernel(with_count: bool):
    return _make_seg_sum(with_count)


def _seg_sum_cnt(vals, src, dst):
    return _get_seg_kernel(True)(vals, src, dst)


def _seg_sum(vals, src, dst):
    return _get_seg_kernel(False)(vals, src, dst)


def _node_phase(x_row, acc_p, cnt_p, ws1c, wn1c, b1c, ws2c, wn2c):
    def body(x_ref, acc_ref, cnt_ref, ws1_ref, wn1_ref, b1_ref, ws2_ref,
             wn2_ref, s_ref, u_ref, deg_ref):
        deg = jnp.maximum(jnp.sum(cnt_ref[...], axis=0, keepdims=True), 1.0)
        m1 = jnp.sum(acc_ref[...], axis=0, keepdims=True) / deg
        h1 = jnp.tanh(ws1_ref[...] * x_ref[...] + wn1_ref[...] * m1
                      + b1_ref[...])
        s_ref[...] = jnp.dot(ws2_ref[...], h1,
                             preferred_element_type=jnp.float32)
        u_ref[...] = jnp.dot(wn2_ref[...], h1,
                             preferred_element_type=jnp.float32)
        deg_ref[...] = deg

    return pl.pallas_call(
        body,
        out_shape=[jax.ShapeDtypeStruct((1, N), jnp.float32),
                   jax.ShapeDtypeStruct((1, N), jnp.float32),
                   jax.ShapeDtypeStruct((1, N), jnp.float32)],
    )(x_row, acc_p, cnt_p, ws1c, wn1c, b1c, ws2c, wn2c)


def _o1_matvec(s, sum2_p, deg, b2, w3, b3_row, block_rows=200):
    nsteps = N // block_rows
    s3 = s.reshape(nsteps, 1, block_rows)
    deg3 = deg.reshape(nsteps, 1, block_rows)
    p3 = sum2_p.reshape(NW, nsteps, block_rows).transpose(1, 0, 2)

    def body(s_ref, p_ref, deg_ref, b2_ref, w3_ref, b3_ref, o1_ref, o2_ref):
        i = pl.program_id(0)
        p = p_ref[...].reshape(NW, block_rows)
        m2 = jnp.sum(p, axis=0, keepdims=True) \
            / deg_ref[...].reshape(1, block_rows)
        o1b = s_ref[...].reshape(1, block_rows) + m2 + b2_ref[0, 0]
        o1_ref[...] = o1b.reshape(1, 1, block_rows)
        h2 = jnp.tanh(o1b)

        @pl.when(i == 0)
        def _():
            o2_ref[...] = b3_ref[...]

        o2_ref[...] += jnp.dot(h2, w3_ref[...],
                               preferred_element_type=jnp.float32)

    o1_3d, o2 = pl.pallas_call(
        body,
        grid=(nsteps,),
        in_specs=[
            pl.BlockSpec((1, 1, block_rows), lambda i: (i, 0, 0)),
            pl.BlockSpec((1, NW, block_rows), lambda i: (i, 0, 0)),
            pl.BlockSpec((1, 1, block_rows), lambda i: (i, 0, 0)),
            pl.BlockSpec(memory_space=pltpu.SMEM),
            pl.BlockSpec((block_rows, N), lambda i: (i, 0)),
            pl.BlockSpec((1, N), lambda i: (0, 0)),
        ],
        out_specs=[pl.BlockSpec((1, 1, block_rows), lambda i: (i, 0, 0)),
                   pl.BlockSpec((1, N), lambda i: (0, 0))],
        out_shape=[jax.ShapeDtypeStruct((nsteps, 1, block_rows), jnp.float32),
                   jax.ShapeDtypeStruct((1, N), jnp.float32)],
    )(s3, p3, deg3, b2, w3, b3_row)
    return o1_3d.reshape(1, N), o2


def kernel(node_features, edge_index, w_self1, w_neigh1, b1, w_self2,
           w_neigh2, b2, w3, b3):
    x_flat = node_features.reshape(N)
    src = edge_index[0]
    dst = edge_index[1]

    sum1_p, cnt_p = _seg_sum_cnt(x_flat, src, dst)

    x_row = node_features.reshape(1, N)
    ws1c = w_self1.reshape(H, 1)
    wn1c = w_neigh1.reshape(H, 1)
    b1c = b1.reshape(H, 1)
    ws2c = w_self2.reshape(1, H)
    wn2c = w_neigh2.reshape(1, H)
    s, u, deg = _node_phase(x_row, sum1_p, cnt_p, ws1c, wn1c, b1c, ws2c, wn2c)

    (sum2_p,) = _seg_sum(u.reshape(N), src, dst)

    o1, o2 = _o1_matvec(s, sum2_p, deg, b2.reshape(1, 1), w3,
                        b3.reshape(1, N))
    return (o1, o2)


# speedup vs baseline: 2.9191x; 1.0846x over previous
import functools

import jax
import jax.numpy as jnp
from jax import lax
from jax.experimental import pallas as pl
from jax.experimental.pallas import tpu as pltpu
from jax.experimental.pallas import tpu_sc as plsc

N = 10000
E = 640000
H = 128
NC, NS = 2, 16
NW = NC * NS
EPW = E // NW
L = 16

_SC_MESH = dict(core_axis_name="c", subcore_axis_name="s",
                num_cores=NC, num_subcores=NS)


def _make_seg_sum(with_count: bool):
    out_type = [jax.ShapeDtypeStruct((NW, N), jnp.float32)]
    scratch = [
        pltpu.VMEM((N,), jnp.float32),
        pltpu.VMEM((EPW,), jnp.int32),
        pltpu.VMEM((EPW,), jnp.int32),
        pltpu.VMEM((N,), jnp.float32),
    ]
    if with_count:
        out_type.append(jax.ShapeDtypeStruct((NW, N), jnp.float32))
        scratch.append(pltpu.VMEM((N,), jnp.float32))
    scratch += [pltpu.SemaphoreType.DMA] * 3

    UNROLL = 10
    assert (EPW // L) % UNROLL == 0

    @functools.partial(
        pl.kernel,
        out_type=out_type,
        mesh=plsc.VectorSubcoreMesh(**_SC_MESH),
        scratch_types=scratch,
        compiler_params=pltpu.CompilerParams(needs_layout_passes=False),
    )
    def seg(vals_hbm, src_hbm, dst_hbm, acc_out, *rest):
        if with_count:
            cnt_out, vals_v, src_v, dst_v, acc_v, cnt_v, s0, s1, s2 = rest
        else:
            vals_v, src_v, dst_v, acc_v, s0, s1, s2 = rest
        wid = lax.axis_index("s") * NC + lax.axis_index("c")
        base = wid * EPW
        c0 = pltpu.async_copy(vals_hbm, vals_v, s0)
        c1 = pltpu.async_copy(src_hbm.at[pl.ds(base, EPW)], src_v, s1)
        c2 = pltpu.async_copy(dst_hbm.at[pl.ds(base, EPW)], dst_v, s2)

        zeros = jnp.zeros((L,), jnp.float32)

        def zbody(i):
            acc_v[pl.ds(i * L, L)] = zeros
            if with_count:
                cnt_v[pl.ds(i * L, L)] = zeros

        plsc.parallel_loop(0, N // L, unroll=5)(zbody)
        c0.wait()
        c1.wait()
        c2.wait()

        ones = jnp.ones((L,), jnp.float32)

        def ebody(i):
            si = src_v[pl.ds(i * L, L)]
            di = dst_v[pl.ds(i * L, L)]
            v = plsc.load_gather(vals_v, [si])
            plsc.addupdate_scatter(acc_v, [di], v)
            if with_count:
                plsc.addupdate_scatter(cnt_v, [di], ones)

        plsc.parallel_loop(0, EPW // L, unroll=UNROLL)(ebody)

        pltpu.sync_copy(acc_v, acc_out.at[wid])
        if with_count:
            pltpu.sync_copy(cnt_v, cnt_out.at[wid])

    return seg


@functools.lru_cache(maxsize=None)
def _get_seg_kernel(with_count: bool):
    return _make_seg_sum(with_count)


def _seg_sum_cnt(vals, src, dst):
    return _get_seg_kernel(True)(vals, src, dst)


def _seg_sum(vals, src, dst):
    return _get_seg_kernel(False)(vals, src, dst)


def _node_phase(x_row, acc_p, cnt_p, ws1c, wn1c, b1c, ws2c, wn2c):
    def body(x_ref, acc_ref, cnt_ref, ws1_ref, wn1_ref, b1_ref, ws2_ref,
             wn2_ref, s_ref, u_ref, deg_ref):
        deg = jnp.maximum(jnp.sum(cnt_ref[...], axis=0, keepdims=True), 1.0)
        m1 = jnp.sum(acc_ref[...], axis=0, keepdims=True) / deg
        h1 = jnp.tanh(ws1_ref[...] * x_ref[...] + wn1_ref[...] * m1
                      + b1_ref[...])
        s_ref[...] = jnp.dot(ws2_ref[...], h1,
                             preferred_element_type=jnp.float32)
        u_ref[...] = jnp.dot(wn2_ref[...], h1,
                             preferred_element_type=jnp.float32)
        deg_ref[...] = deg

    return pl.pallas_call(
        body,
        out_shape=[jax.ShapeDtypeStruct((1, N), jnp.float32),
                   jax.ShapeDtypeStruct((1, N), jnp.float32),
                   jax.ShapeDtypeStruct((1, N), jnp.float32)],
    )(x_row, acc_p, cnt_p, ws1c, wn1c, b1c, ws2c, wn2c)


def _o1_matvec(s, sum2_p, deg, b2, w3, b3_row, block_rows=200):
    nsteps = N // block_rows
    s3 = s.reshape(nsteps, 1, block_rows)
    deg3 = deg.reshape(nsteps, 1, block_rows)
    p3 = sum2_p.reshape(NW, nsteps, block_rows).transpose(1, 0, 2)

    def body(s_ref, p_ref, deg_ref, b2_ref, w3_ref, b3_ref, o1_ref, o2_ref):
        i = pl.program_id(0)
        p = p_ref[...].reshape(NW, block_rows)
        m2 = jnp.sum(p, axis=0, keepdims=True) \
            / deg_ref[...].reshape(1, block_rows)
        o1b = s_ref[...].reshape(1, block_rows) + m2 + b2_ref[0, 0]
        o1_ref[...] = o1b.reshape(1, 1, block_rows)
        h2 = jnp.tanh(o1b)

        @pl.when(i == 0)
        def _():
            o2_ref[...] = b3_ref[...]

        o2_ref[...] += jnp.dot(h2, w3_ref[...],
                               preferred_element_type=jnp.float32)

    o1_3d, o2 = pl.pallas_call(
        body,
        grid=(nsteps,),
        in_specs=[
            pl.BlockSpec((1, 1, block_rows), lambda i: (i, 0, 0)),
            pl.BlockSpec((1, NW, block_rows), lambda i: (i, 0, 0)),
            pl.BlockSpec((1, 1, block_rows), lambda i: (i, 0, 0)),
            pl.BlockSpec(memory_space=pltpu.SMEM),
            pl.BlockSpec((block_rows, N), lambda i: (i, 0)),
            pl.BlockSpec((1, N), lambda i: (0, 0)),
        ],
        out_specs=[pl.BlockSpec((1, 1, block_rows), lambda i: (i, 0, 0)),
                   pl.BlockSpec((1, N), lambda i: (0, 0))],
        out_shape=[jax.ShapeDtypeStruct((nsteps, 1, block_rows), jnp.float32),
                   jax.ShapeDtypeStruct((1, N), jnp.float32)],
    )(s3, p3, deg3, b2, w3, b3_row)
    return o1_3d.reshape(1, N), o2


def kernel(node_features, edge_index, w_self1, w_neigh1, b1, w_self2,
           w_neigh2, b2, w3, b3):
    x_flat = node_features.reshape(N)
    src = edge_index[0]
    dst = edge_index[1]

    sum1_p, cnt_p = _seg_sum_cnt(x_flat, src, dst)

    x_row = node_features.reshape(1, N)
    ws1c = w_self1.reshape(H, 1)
    wn1c = w_neigh1.reshape(H, 1)
    b1c = b1.reshape(H, 1)
    ws2c = w_self2.reshape(1, H)
    wn2c = w_neigh2.reshape(1, H)
    s, u, deg = _node_phase(x_row, sum1_p, cnt_p, ws1c, wn1c, b1c, ws2c, wn2c)

    (sum2_p,) = _seg_sum(u.reshape(N), src, dst)

    o1, o2 = _o1_matvec(s, sum2_p, deg, b2.reshape(1, 1), w3,
                        b3.reshape(1, N))
    return (o1, o2)
